# baseline (speedup 1.0000x reference)
import functools

import jax
import jax.numpy as jnp
from jax import lax
from jax.experimental import pallas as pl
from jax.experimental.pallas import tpu as pltpu
from jax.experimental.pallas import tpu_sc as plsc

_NC = 2
_NS = 16
_NW = _NC * _NS
_L = 16

_LN2 = 0.6931471805599453
_SQRT2 = 1.4142135623730951


def _vlog(v):
    xi = lax.bitcast_convert_type(v, jnp.int32)
    e = ((xi >> 23) & 0xFF) - 127
    m = lax.bitcast_convert_type((xi & 0x7FFFFF) | 0x3F800000, jnp.float32)
    big = m > _SQRT2
    m = jnp.where(big, m * 0.5, m)
    e = jnp.where(big, e + 1, e)
    z = (m - 1.0) / (m + 1.0)
    z2 = z * z
    p = z * (2.0 + z2 * (2.0 / 3.0 + z2 * (0.4 + z2 * (2.0 / 7.0))))
    return e.astype(jnp.float32) * _LN2 + p


def _sc_body(rows_w, b_w, emb,
             idx_hbm, x0_hbm, x1_hbm, xu0_hbm, xu1_hbm, pp_hbm, mgb_hbm,
             bias_flat_hbm, ent_hbm,
             pred_out, klb_out, kle_out,
             idxv, i2a, i2b, ev, bmv, bsv, klev, klbv,
             xv0, xv1, cidx0, cidx1, c2a, c2b,
             e0, e1, b0m, b1m, predv, ppv, mgbv,
             sem_k, sem_c, sem_p):
    wid = lax.axis_index("s") * _NC + lax.axis_index("c")
    grp = wid // (_NW // 2)
    kbase = wid * rows_w
    pbase = wid * b_w
    iota = lax.iota(jnp.int32, _L)
    zeros = iota * 0

    pltpu.sync_copy(idx_hbm.at[pl.ds(kbase, rows_w)], idxv)

    def dbl_body(j, carry):
        c = idxv[pl.ds(j * _L, _L)] * 2
        i2a[pl.ds(j * _L, _L)] = c
        i2b[pl.ds(j * _L, _L)] = c + 1
        return carry

    lax.fori_loop(0, rows_w // _L, dbl_body, 0)

    cp_e = pltpu.async_copy(ent_hbm.at[idxv], ev, sem_k)
    cp_m = pltpu.async_copy(bias_flat_hbm.at[i2a], bmv, sem_k)
    cp_s = pltpu.async_copy(bias_flat_hbm.at[i2b], bsv, sem_k)

    pltpu.sync_copy(x0_hbm.at[pl.ds(pbase, b_w)], xv0)
    pltpu.sync_copy(x1_hbm.at[pl.ds(pbase, b_w)], xv1)
    cc0 = pltpu.async_copy(xu0_hbm.at[xv0], cidx0, sem_c)
    cc1 = pltpu.async_copy(xu1_hbm.at[xv1], cidx1, sem_c)
    pltpu.sync_copy(pp_hbm.at[grp], ppv)
    pltpu.sync_copy(mgb_hbm, mgbv)
    cc0.wait()
    cc1.wait()

    def cdbl_body(j, carry):
        c2a[pl.ds(j * _L, _L)] = cidx0[pl.ds(j * _L, _L)] * 2
        c2b[pl.ds(j * _L, _L)] = cidx1[pl.ds(j * _L, _L)] * 2
        return carry

    lax.fori_loop(0, b_w // _L, cdbl_body, 0)

    ce0 = pltpu.async_copy(ent_hbm.at[cidx0], e0, sem_p)
    ce1 = pltpu.async_copy(ent_hbm.at[cidx1], e1, sem_p)
    cb0 = pltpu.async_copy(bias_flat_hbm.at[c2a], b0m, sem_p)
    cb1 = pltpu.async_copy(bias_flat_hbm.at[c2b], b1m, sem_p)

    bm2 = ppv[0, :]
    bs2 = jnp.abs(ppv[1, :])
    em2 = ppv[2, :]
    es2 = jnp.abs(ppv[3, :])
    log_bs2 = _vlog(bs2)
    inv2bs2 = 0.5 / (bs2 * bs2)
    log_es2 = _vlog(es2)
    inv2es2 = 0.5 / (es2 * es2)

    cp_e.wait()

    def kle_body(i, carry):
        m1 = ev[i, 0:emb]
        s1 = jnp.abs(ev[i, emb:2 * emb])
        d = m1 - em2
        klev[i, :] = log_es2 - _vlog(s1) + (s1 * s1 + d * d) * inv2es2 - 0.5
        return carry

    lax.fori_loop(0, rows_w, kle_body, 0)

    cp_m.wait()
    cp_s.wait()

    def klb_body(j, carry):
        m1 = bmv[pl.ds(j * _L, _L)]
        s1 = jnp.abs(bsv[pl.ds(j * _L, _L)])
        d = m1 - bm2
        klbv[pl.ds(j * _L, _L)] = (
            log_bs2 - _vlog(s1) + (s1 * s1 + d * d) * inv2bs2 - 0.5)
        return carry

    lax.fori_loop(0, rows_w // _L, klb_body, 0)

    pltpu.sync_copy(klev, kle_out.at[pl.ds(kbase, rows_w)])
    pltpu.sync_copy(klbv, klb_out.at[pl.ds(kbase, rows_w)])

    ce0.wait()
    ce1.wait()
    cb0.wait()
    cb1.wait()
    mgv = mgbv[...]

    def pred_body(j, carry):
        r = j * _L + iota
        acc = mgv + b0m[pl.ds(j * _L, _L)] + b1m[pl.ds(j * _L, _L)]
        for k in range(emb):
            acc = acc + plsc.load_gather(e0, [r, zeros + k]) * plsc.load_gather(
                e1, [r, zeros + k])
        predv[pl.ds(j * _L, _L)] = acc
        return carry

    lax.fori_loop(0, b_w // _L, pred_body, 0)
    pltpu.sync_copy(predv, pred_out.at[pl.ds(pbase, b_w)])


@jax.jit
def _cf_sc(idx_flat, x0, x1, xu0, xu1, prior_pack, mgbv, bias_flat,
           entity_params):
    two_u = idx_flat.shape[0]
    b = x0.shape[0]
    emb = entity_params.shape[1] // 2
    rows_w = two_u // _NW
    b_w = b // _NW
    mesh = plsc.VectorSubcoreMesh(
        core_axis_name="c", subcore_axis_name="s", num_cores=_NC)
    body = functools.partial(_sc_body, rows_w, b_w, emb)
    fn = pl.kernel(
        body,
        out_type=(
            jax.ShapeDtypeStruct((b,), jnp.float32),
            jax.ShapeDtypeStruct((two_u,), jnp.float32),
            jax.ShapeDtypeStruct((two_u, emb), jnp.float32),
        ),
        mesh=mesh,
        compiler_params=pltpu.CompilerParams(
            needs_layout_passes=False, use_tc_tiling_on_sc=False),
        scratch_types=[
            pltpu.VMEM((rows_w,), jnp.int32),
            pltpu.VMEM((rows_w,), jnp.int32),
            pltpu.VMEM((rows_w,), jnp.int32),
            pltpu.VMEM((rows_w, 2 * emb), jnp.float32),
            pltpu.VMEM((rows_w,), jnp.float32),
            pltpu.VMEM((rows_w,), jnp.float32),
            pltpu.VMEM((rows_w, emb), jnp.float32),
            pltpu.VMEM((rows_w,), jnp.float32),
            pltpu.VMEM((b_w,), jnp.int32),
            pltpu.VMEM((b_w,), jnp.int32),
            pltpu.VMEM((b_w,), jnp.int32),
            pltpu.VMEM((b_w,), jnp.int32),
            pltpu.VMEM((b_w,), jnp.int32),
            pltpu.VMEM((b_w,), jnp.int32),
            pltpu.VMEM((b_w, 2 * emb), jnp.float32),
            pltpu.VMEM((b_w, 2 * emb), jnp.float32),
            pltpu.VMEM((b_w,), jnp.float32),
            pltpu.VMEM((b_w,), jnp.float32),
            pltpu.VMEM((b_w,), jnp.float32),
            pltpu.VMEM((4, _L), jnp.float32),
            pltpu.VMEM((_L,), jnp.float32),
            pltpu.SemaphoreType.DMA,
            pltpu.SemaphoreType.DMA,
            pltpu.SemaphoreType.DMA,
        ],
    )
    return fn(idx_flat, x0, x1, xu0, xu1, prior_pack, mgbv, bias_flat,
              entity_params)


def kernel(x, x_unique, alpha, mean_global_bias, scale_global_bias,
           bias_params, entity_params, mean_global_bias_prior,
           scale_global_bias_prior, mean_group_bias_prior,
           scale_group_bias_prior, mean_group_entity_prior,
           scale_group_entity_prior):
    emb = entity_params.shape[1] // 2
    u = x_unique.shape[1]

    idx_flat = jnp.reshape(x_unique, (2 * u,))
    prior_pack = jnp.stack([
        jnp.broadcast_to(mean_group_bias_prior, (2, emb)),
        jnp.broadcast_to(scale_group_bias_prior, (2, emb)),
        mean_group_entity_prior,
        scale_group_entity_prior,
    ], axis=1).astype(jnp.float32)
    mgbv = jnp.broadcast_to(mean_global_bias, (_L,)).astype(jnp.float32)
    bias_flat = jnp.reshape(bias_params.astype(jnp.float32), (-1,))

    unscaled_pred, kl_bias, kl_entity = _cf_sc(
        idx_flat, x[0], x[1], x_unique[0], x_unique[1], prior_pack, mgbv,
        bias_flat, entity_params.astype(jnp.float32))

    std_dev = jnp.sqrt(1.0 / jnp.abs(alpha))
    s1g = jnp.abs(scale_global_bias)
    s2g = jnp.abs(scale_global_bias_prior)
    kl_global = (jnp.log(s2g / s1g)
                 + (s1g ** 2 + (mean_global_bias - mean_global_bias_prior) ** 2)
                 / (2.0 * s2g ** 2) - 0.5)
    return (unscaled_pred, std_dev, kl_global, kl_bias, kl_entity)


# speedup vs baseline: 2.1373x; 2.1373x over previous
import functools

import jax
import jax.numpy as jnp
from jax import lax
from jax.experimental import pallas as pl
from jax.experimental.pallas import tpu as pltpu
from jax.experimental.pallas import tpu_sc as plsc

_NC = 2
_NS = 16
_NW = _NC * _NS
_L = 16

_LN2 = 0.6931471805599453
_SQRT2 = 1.4142135623730951


def _vlog(v):
    xi = lax.bitcast_convert_type(v, jnp.int32)
    e = ((xi >> 23) & 0xFF) - 127
    m = lax.bitcast_convert_type((xi & 0x7FFFFF) | 0x3F800000, jnp.float32)
    big = m > _SQRT2
    m = jnp.where(big, m * 0.5, m)
    e = jnp.where(big, e + 1, e)
    z = (m - 1.0) / (m + 1.0)
    z2 = z * z
    p = z * (2.0 + z2 * (2.0 / 3.0 + z2 * (0.4 + z2 * (2.0 / 7.0))))
    return e.astype(jnp.float32) * _LN2 + p


def _sc_body(rows_w, b_w, emb, v_rows,
             idx_hbm, x0_hbm, x1_hbm, xu0_hbm, xu1_hbm,
             pb_hbm, em2_hbm, es2_hbm, mgb_hbm, bias_t_hbm, ent_hbm,
             pred_out, klb_out, kle_out,
             idxv, kval, bmv, bsv, klbv,
             xv0, xv1, cidx0, cidx1,
             e0t, e1t, b0m, b1m, predv,
             pbv, em2v, es2v, mgv,
             sem_k, sem_b, sem_c, sem_p, sem_o):
    wid = lax.axis_index("s") * _NC + lax.axis_index("c")
    grp = wid // (_NW // 2)
    kbase = wid * rows_w
    pbase = wid * b_w

    pltpu.sync_copy(idx_hbm.at[pl.ds(kbase, rows_w)], idxv)

    kl_cps = []
    for k in range(2 * emb):
        kl_cps.append(pltpu.async_copy(
            ent_hbm.at[pl.ds(k * v_rows, v_rows)].at[idxv], kval.at[k], sem_k))
    cb_m = pltpu.async_copy(bias_t_hbm.at[pl.ds(0, v_rows)].at[idxv], bmv, sem_b)
    cb_s = pltpu.async_copy(
        bias_t_hbm.at[pl.ds(v_rows, v_rows)].at[idxv], bsv, sem_b)

    pltpu.sync_copy(x0_hbm.at[pl.ds(pbase, b_w)], xv0)
    pltpu.sync_copy(x1_hbm.at[pl.ds(pbase, b_w)], xv1)
    cc0 = pltpu.async_copy(xu0_hbm.at[xv0], cidx0, sem_c)
    cc1 = pltpu.async_copy(xu1_hbm.at[xv1], cidx1, sem_c)

    pltpu.sync_copy(pb_hbm.at[grp], pbv)
    pltpu.sync_copy(em2_hbm.at[grp], em2v)
    pltpu.sync_copy(es2_hbm.at[grp], es2v)
    pltpu.sync_copy(mgb_hbm, mgv)

    cc0.wait()
    cc1.wait()
    p_cps = []
    for k in range(emb):
        p_cps.append(pltpu.async_copy(
            ent_hbm.at[pl.ds(k * v_rows, v_rows)].at[cidx0], e0t.at[k], sem_p))
        p_cps.append(pltpu.async_copy(
            ent_hbm.at[pl.ds(k * v_rows, v_rows)].at[cidx1], e1t.at[k], sem_p))
    p_cps.append(pltpu.async_copy(
        bias_t_hbm.at[pl.ds(0, v_rows)].at[cidx0], b0m, sem_p))
    p_cps.append(pltpu.async_copy(
        bias_t_hbm.at[pl.ds(0, v_rows)].at[cidx1], b1m, sem_p))

    bm2 = pbv[0, :]
    bs2 = jnp.abs(pbv[1, :])
    log_bs2 = _vlog(bs2)
    inv2bs2 = 0.5 / (bs2 * bs2)
    cb_m.wait()
    cb_s.wait()

    def klb_body(j, carry):
        m1 = bmv[pl.ds(j * _L, _L)]
        s1 = jnp.abs(bsv[pl.ds(j * _L, _L)])
        d = m1 - bm2
        klbv[pl.ds(j * _L, _L)] = (
            log_bs2 - _vlog(s1) + (s1 * s1 + d * d) * inv2bs2 - 0.5)
        return carry

    lax.fori_loop(0, rows_w // _L, klb_body, 0)
    out_klb = pltpu.async_copy(klbv, klb_out.at[pl.ds(kbase, rows_w)], sem_o)

    for cp in kl_cps:
        cp.wait()

    out_kle = []
    for k in range(emb):
        em2k = em2v[k, :]
        es2k = jnp.abs(es2v[k, :])
        log_es2k = _vlog(es2k)
        inv2es2k = 0.5 / (es2k * es2k)

        def kle_body(j, carry, k=k, em2k=em2k, log_es2k=log_es2k,
                     inv2es2k=inv2es2k):
            m1 = kval[k, pl.ds(j * _L, _L)]
            s1 = jnp.abs(kval[emb + k, pl.ds(j * _L, _L)])
            d = m1 - em2k
            kval[k, pl.ds(j * _L, _L)] = (
                log_es2k - _vlog(s1) + (s1 * s1 + d * d) * inv2es2k - 0.5)
            return carry

        lax.fori_loop(0, rows_w // _L, kle_body, 0)
        out_kle.append(pltpu.async_copy(
            kval.at[k], kle_out.at[k, pl.ds(kbase, rows_w)], sem_o))

    for cp in p_cps:
        cp.wait()
    mg = mgv[...]

    def pred_body(j, carry):
        sl = pl.ds(j * _L, _L)
        acc = mg + b0m[sl] + b1m[sl]
        for k in range(emb):
            acc = acc + e0t[k, sl] * e1t[k, sl]
        predv[sl] = acc
        return carry

    lax.fori_loop(0, b_w // _L, pred_body, 0)
    pltpu.sync_copy(predv, pred_out.at[pl.ds(pbase, b_w)])

    out_klb.wait()
    for cp in out_kle:
        cp.wait()


@jax.jit
def _cf_sc(idx_flat, x0, x1, xu0, xu1, pb, em2_bc, es2_bc, mgbv, bias_t,
           ent_flat):
    two_u = idx_flat.shape[0]
    b = x0.shape[0]
    emb = em2_bc.shape[1]
    v_rows = bias_t.shape[0] // 2
    rows_w = two_u // _NW
    b_w = b // _NW
    mesh = plsc.VectorSubcoreMesh(
        core_axis_name="c", subcore_axis_name="s", num_cores=_NC)
    body = functools.partial(_sc_body, rows_w, b_w, emb, v_rows)
    fn = pl.kernel(
        body,
        out_type=(
            jax.ShapeDtypeStruct((b,), jnp.float32),
            jax.ShapeDtypeStruct((two_u,), jnp.float32),
            jax.ShapeDtypeStruct((emb, two_u), jnp.float32),
        ),
        mesh=mesh,
        compiler_params=pltpu.CompilerParams(
            needs_layout_passes=False, use_tc_tiling_on_sc=False),
        scratch_types=[
            pltpu.VMEM((rows_w,), jnp.int32),
            pltpu.VMEM((2 * emb, rows_w), jnp.float32),
            pltpu.VMEM((rows_w,), jnp.float32),
            pltpu.VMEM((rows_w,), jnp.float32),
            pltpu.VMEM((rows_w,), jnp.float32),
            pltpu.VMEM((b_w,), jnp.int32),
            pltpu.VMEM((b_w,), jnp.int32),
            pltpu.VMEM((b_w,), jnp.int32),
            pltpu.VMEM((b_w,), jnp.int32),
            pltpu.VMEM((emb, b_w), jnp.float32),
            pltpu.VMEM((emb, b_w), jnp.float32),
            pltpu.VMEM((b_w,), jnp.float32),
            pltpu.VMEM((b_w,), jnp.float32),
            pltpu.VMEM((b_w,), jnp.float32),
            pltpu.VMEM((2, _L), jnp.float32),
            pltpu.VMEM((_L, _L), jnp.float32),
            pltpu.VMEM((_L, _L), jnp.float32),
            pltpu.VMEM((_L,), jnp.float32),
            pltpu.SemaphoreType.DMA,
            pltpu.SemaphoreType.DMA,
            pltpu.SemaphoreType.DMA,
            pltpu.SemaphoreType.DMA,
            pltpu.SemaphoreType.DMA,
        ],
    )
    return fn(idx_flat, x0, x1, xu0, xu1, pb, em2_bc, es2_bc, mgbv, bias_t,
              ent_flat)


def kernel(x, x_unique, alpha, mean_global_bias, scale_global_bias,
           bias_params, entity_params, mean_global_bias_prior,
           scale_global_bias_prior, mean_group_bias_prior,
           scale_group_bias_prior, mean_group_entity_prior,
           scale_group_entity_prior):
    emb = entity_params.shape[1] // 2
    u = x_unique.shape[1]

    idx_flat = jnp.reshape(x_unique, (2 * u,))
    ent_flat = jnp.reshape(jnp.transpose(entity_params.astype(jnp.float32)),
                           (-1,))
    bias_t = jnp.reshape(jnp.transpose(bias_params.astype(jnp.float32)),
                         (-1,))

    pb = jnp.stack([
        jnp.broadcast_to(mean_group_bias_prior, (2, _L)),
        jnp.broadcast_to(scale_group_bias_prior, (2, _L)),
    ], axis=1).astype(jnp.float32)
    em2_bc = jnp.broadcast_to(
        mean_group_entity_prior.astype(jnp.float32)[:, :, None], (2, emb, _L))
    es2_bc = jnp.broadcast_to(
        scale_group_entity_prior.astype(jnp.float32)[:, :, None], (2, emb, _L))
    mgbv = jnp.broadcast_to(mean_global_bias, (_L,)).astype(jnp.float32)

    unscaled_pred, kl_bias, kle_t = _cf_sc(
        idx_flat, x[0], x[1], x_unique[0], x_unique[1], pb, em2_bc, es2_bc,
        mgbv, bias_t, ent_flat)
    kl_entity = jnp.transpose(kle_t)

    std_dev = jnp.sqrt(1.0 / jnp.abs(alpha))
    s1g = jnp.abs(scale_global_bias)
    s2g = jnp.abs(scale_global_bias_prior)
    kl_global = (jnp.log(s2g / s1g)
                 + (s1g ** 2 + (mean_global_bias - mean_global_bias_prior) ** 2)
                 / (2.0 * s2g ** 2) - 0.5)
    return (unscaled_pred, std_dev, kl_global, kl_bias, kl_entity)
